# baseline (speedup 1.0000x reference)
import jax
import jax.numpy as jnp
from jax import lax
from jax.experimental import pallas as pl
from jax.experimental.pallas import tpu as pltpu
from jax.experimental.pallas import tpu_sc as plsc

VOCAB = 1000000
BATCH = 16384
TOKENS = 20
D = 64
DPAD = 128
NC, NS = 2, 16
NW = NC * NS
DOCS_PER_W = BATCH // NW
CHUNK_DOCS = 32
ROWS_PER_CHUNK = CHUNK_DOCS * TOKENS
GATHERS_PER_CHUNK = ROWS_PER_CHUNK // 16
CHUNKS = DOCS_PER_W // CHUNK_DOCS
STAGE_COLS = 128

PACK_BLK = 8000


def _pack_kernel(t_ref, w_ref, o_ref):
    y = lax.dot_general(
        t_ref[...],
        w_ref[...],
        (((1,), (1,)), ((), ())),
        preferred_element_type=jnp.float32,
        precision=lax.Precision.HIGHEST,
    )
    o_ref[:, pl.ds(0, D)] = y
    o_ref[:, pl.ds(D, D)] = y


def _pack_tc(table, W):
    return pl.pallas_call(
        _pack_kernel,
        out_shape=jax.ShapeDtypeStruct((VOCAB, DPAD), jnp.float32),
        grid=(VOCAB // PACK_BLK,),
        in_specs=[
            pl.BlockSpec((PACK_BLK, D), lambda i: (i, 0)),
            pl.BlockSpec((D, D), lambda i: (0, 0)),
        ],
        out_specs=pl.BlockSpec((PACK_BLK, DPAD), lambda i: (i, 0)),
    )(table, W)


def _pool_sc_kernel(
    doc_hbm, table2_hbm, out_hbm, idx_v, idx_c, rows_v, out_v, sem
):
    wid = lax.axis_index("s") * NC + lax.axis_index("c")

    @pl.loop(0, CHUNKS)
    def _chunk(c):
        g = wid * CHUNKS + c
        doc_base = g * CHUNK_DOCS
        pltpu.sync_copy(
            doc_hbm.at[pl.ds(doc_base, CHUNK_DOCS), pl.ds(0, STAGE_COLS)],
            idx_v,
        )
        lane = lax.broadcasted_iota(jnp.int32, (16,), 0)
        for j in range(ROWS_PER_CHUNK // 16):
            f = lane + (16 * j)
            dv = lax.shift_right_logical(f * 52429, 20)
            tv = f - dv * TOKENS
            iv = plsc.load_gather(idx_v, [dv, tv])
            idx_c[j // 8, pl.ds(16 * (j % 8), 16)] = iv
        descs = []
        for r in range(ROWS_PER_CHUNK // 128):
            descs.append(
                pltpu.async_copy(
                    table2_hbm.at[idx_c.at[r]],
                    rows_v.at[pl.ds(128 * r, 128)],
                    sem,
                )
            )
        for desc in descs:
            desc.wait()

        @pl.loop(0, CHUNK_DOCS)
        def _doc(d):
            row0 = d * TOKENS

            def body(t, accs):
                return tuple(
                    accs[k] + rows_v[row0 + t, pl.ds(k * 16, 16)]
                    for k in range(4)
                )

            zero = jnp.zeros((16,), jnp.float32)
            accs = lax.fori_loop(0, TOKENS, body, (zero, zero, zero, zero))
            for k in range(4):
                out_v[d, pl.ds(k * 16, 16)] = accs[k]

        pltpu.sync_copy(out_v, out_hbm.at[pl.ds(doc_base, CHUNK_DOCS)])


def _pool_sc(document, table2):
    mesh = plsc.VectorSubcoreMesh(
        core_axis_name="c", subcore_axis_name="s", num_cores=NC, num_subcores=NS
    )
    f = pl.kernel(
        _pool_sc_kernel,
        out_type=jax.ShapeDtypeStruct((BATCH, D), jnp.float32),
        mesh=mesh,
        scratch_types=[
            pltpu.VMEM((CHUNK_DOCS, STAGE_COLS), jnp.int32),
            pltpu.VMEM((ROWS_PER_CHUNK // 128, 128), jnp.int32),
            pltpu.VMEM((ROWS_PER_CHUNK, DPAD), jnp.float32),
            pltpu.VMEM((CHUNK_DOCS, D), jnp.float32),
            pltpu.SemaphoreType.DMA,
        ],
        compiler_params=pltpu.CompilerParams(needs_layout_passes=False),
    )
    return f(document, table2)


def kernel(document, table, W):
    table2 = _pack_tc(table, W)
    return _pool_sc(document, table2)


# speedup vs baseline: 1.0052x; 1.0052x over previous
import jax
import jax.numpy as jnp
from jax import lax
from jax.experimental import pallas as pl
from jax.experimental.pallas import tpu as pltpu
from jax.experimental.pallas import tpu_sc as plsc

VOCAB = 1000000
BATCH = 16384
TOKENS = 20
D = 64
DPAD = 128
NC, NS = 2, 16
NW = NC * NS
DOCS_PER_W = BATCH // NW
CHUNK_DOCS = 32
ROWS_PER_CHUNK = CHUNK_DOCS * TOKENS
GATHERS_PER_CHUNK = ROWS_PER_CHUNK // 128
CHUNKS = DOCS_PER_W // CHUNK_DOCS

PACK_BLK = 8000


def _pack_kernel(t_ref, w_ref, o_ref):
    y = lax.dot_general(
        t_ref[...],
        w_ref[...],
        (((1,), (1,)), ((), ())),
        preferred_element_type=jnp.float32,
        precision=lax.Precision.HIGHEST,
    )
    o_ref[:, pl.ds(0, D)] = y
    o_ref[:, pl.ds(D, D)] = y


def _pack_tc(table, W):
    return pl.pallas_call(
        _pack_kernel,
        out_shape=jax.ShapeDtypeStruct((VOCAB, DPAD), jnp.float32),
        grid=(VOCAB // PACK_BLK,),
        in_specs=[
            pl.BlockSpec((PACK_BLK, D), lambda i: (i, 0)),
            pl.BlockSpec((D, D), lambda i: (0, 0)),
        ],
        out_specs=pl.BlockSpec((PACK_BLK, DPAD), lambda i: (i, 0)),
    )(table, W)


def _pool_sc_kernel(idx_hbm, table2_hbm, out_hbm, idx_c, rows_v, out_v, sem):
    wid = lax.axis_index("s") * NC + lax.axis_index("c")

    @pl.loop(0, CHUNKS)
    def _chunk(c):
        g = wid * CHUNKS + c
        doc_base = g * CHUNK_DOCS
        pltpu.sync_copy(idx_hbm.at[g], idx_c)
        descs = []
        for r in range(GATHERS_PER_CHUNK):
            descs.append(
                pltpu.async_copy(
                    table2_hbm.at[idx_c.at[r]],
                    rows_v.at[pl.ds(128 * r, 128)],
                    sem,
                )
            )
        for desc in descs:
            desc.wait()

        @pl.loop(0, CHUNK_DOCS)
        def _doc(d):
            row0 = d * TOKENS

            def body(t, accs):
                return tuple(
                    accs[k] + rows_v[row0 + t, pl.ds(k * 16, 16)]
                    for k in range(4)
                )

            zero = jnp.zeros((16,), jnp.float32)
            accs = lax.fori_loop(0, TOKENS, body, (zero, zero, zero, zero))
            for k in range(4):
                out_v[d, pl.ds(k * 16, 16)] = accs[k]

        pltpu.sync_copy(out_v, out_hbm.at[pl.ds(doc_base, CHUNK_DOCS)])


def _pool_sc(idx, table2):
    mesh = plsc.VectorSubcoreMesh(
        core_axis_name="c", subcore_axis_name="s", num_cores=NC, num_subcores=NS
    )
    f = pl.kernel(
        _pool_sc_kernel,
        out_type=jax.ShapeDtypeStruct((BATCH, D), jnp.float32),
        mesh=mesh,
        scratch_types=[
            pltpu.VMEM((GATHERS_PER_CHUNK, 128), jnp.int32),
            pltpu.VMEM((ROWS_PER_CHUNK, DPAD), jnp.float32),
            pltpu.VMEM((CHUNK_DOCS, D), jnp.float32),
            pltpu.SemaphoreType.DMA,
        ],
    )
    return f(idx, table2)


def kernel(document, table, W):
    table2 = _pack_tc(table, W)
    idx = document[:, :TOKENS].reshape(NW * CHUNKS, GATHERS_PER_CHUNK, 128)
    return _pool_sc(idx, table2)
